# baseline (speedup 1.0000x reference)
import functools

import jax
import jax.numpy as jnp
from jax import lax
from jax.experimental import pallas as pl
from jax.experimental.pallas import tpu as pltpu
from jax.experimental.pallas import tpu_sc as plsc

_ROWS, _COLS = 16384, 200
_N = _ROWS * _COLS
_NC, _NS = 2, 16
_NW = _NC * _NS
_CB = _COLS // 8
_RB = _ROWS // 128
_RB_W = _RB // _NW

_mesh = plsc.VectorSubcoreMesh(core_axis_name="c", subcore_axis_name="s")


@functools.partial(
    pl.kernel,
    out_type=jax.ShapeDtypeStruct((_COLS, _RB * 4, 128), jnp.float32),
    mesh=_mesh,
    scratch_types=[
        pltpu.VMEM((2, 2, _RB_W, 8, 128), jnp.int32),
        pltpu.VMEM((2, 16, _RB_W * 4, 128), jnp.float32),
        pltpu.VMEM((32,), jnp.float32),
        pltpu.SemaphoreType.DMA,
        pltpu.SemaphoreType.DMA,
        pltpu.SemaphoreType.DMA,
        pltpu.SemaphoreType.DMA,
    ],
    compiler_params=pltpu.CompilerParams(needs_layout_passes=False),
)
def _lookup(idx_hbm, tab_hbm, out_hbm, in_v, out_v, tab_v,
            isem0, isem1, osem0, osem1):
    wid = lax.axis_index("s") * _NC + lax.axis_index("c")
    rb0 = wid * _RB_W
    pltpu.sync_copy(tab_hbm, tab_v)
    isems = (isem0, isem1)
    osems = (osem0, osem1)

    def in_copy(u, b):
        return pltpu.make_async_copy(
            idx_hbm.at[pl.ds(u * 2, 2), pl.ds(rb0, _RB_W)], in_v.at[b],
            isems[b])

    def out_copy(u, b):
        return pltpu.make_async_copy(
            out_v.at[b],
            out_hbm.at[pl.ds(u * 16, 16), pl.ds(rb0 * 4, _RB_W * 4)],
            osems[b])

    def in_copy_tail(b):
        return pltpu.make_async_copy(
            idx_hbm.at[_CB - 1, pl.ds(rb0, _RB_W)], in_v.at[b, 0], isems[b])

    def out_copy_tail(b):
        return pltpu.make_async_copy(
            out_v.at[b, pl.ds(0, 8)],
            out_hbm.at[pl.ds((_CB - 1) * 8, 8), pl.ds(rb0 * 4, _RB_W * 4)],
            osems[b])

    def compute(b, ncb):
        @plsc.parallel_loop(0, ncb * 8 * _RB_W, unroll=4)
        def row_body(i):
            cbl = i // (8 * _RB_W)
            rem = i - cbl * (8 * _RB_W)
            cs = rem // _RB_W
            rbl = rem - cs * _RB_W
            for g in range(8):
                iv = in_v[b, cbl, rbl, cs, pl.ds(g * 16, 16)]
                for j in range(4):
                    col = plsc.load_gather(tab_v, [iv + (6 * j)])
                    out_v[b, cbl * 8 + cs, rbl * 4 + j, pl.ds(g * 16, 16)] = col

    def unit(u, b, skip_out_wait):
        in_copy(u, b).wait()

        @pl.when(jnp.logical_not(skip_out_wait))
        def _():
            out_copy(u, b).wait()

        compute(b, 2)
        out_copy(u, b).start()

    in_copy(0, 0).start()
    _NU = (_CB - 1) // 2

    def pair_body(p, carry):
        u0 = 2 * p
        in_copy(u0 + 1, 1).start()
        unit(u0, 0, p == 0)

        @pl.when(p < _NU // 2 - 1)
        def _():
            in_copy(u0 + 2, 0).start()

        @pl.when(p == _NU // 2 - 1)
        def _():
            in_copy_tail(0).start()

        unit(u0 + 1, 1, p == 0)
        return carry

    lax.fori_loop(0, _NU // 2, pair_body, 0)
    in_copy_tail(0).wait()
    out_copy(_NU - 2, 0).wait()
    compute(0, 1)
    out_copy_tail(0).start()
    out_copy_tail(0).wait()
    out_copy(_NU - 1, 1).wait()


def kernel(inputs, embed_weight):
    idx4 = (
        inputs.astype(jnp.int32)
        .reshape(_RB, 128, _CB, 8)
        .transpose(2, 0, 3, 1)
    )
    tab = jnp.zeros((32,), jnp.float32)
    tab = tab.at[:24].set(embed_weight.astype(jnp.float32).T.reshape(-1))
    out3 = _lookup(idx4, tab)
    return (
        out3.reshape(_COLS, _RB, 4, 128)
        .transpose(1, 3, 0, 2)
        .reshape(_ROWS, _COLS, 4)
    )


# speedup vs baseline: 1.0137x; 1.0137x over previous
import functools

import jax
import jax.numpy as jnp
from jax import lax
from jax.experimental import pallas as pl
from jax.experimental.pallas import tpu as pltpu
from jax.experimental.pallas import tpu_sc as plsc

_ROWS, _COLS = 16384, 200
_N = _ROWS * _COLS
_NC, _NS = 2, 16
_NW = _NC * _NS
_CB = _COLS // 8
_RB = _ROWS // 128
_RB_W = _RB // _NW

_mesh = plsc.VectorSubcoreMesh(core_axis_name="c", subcore_axis_name="s")


@functools.partial(
    pl.kernel,
    out_type=jax.ShapeDtypeStruct((_COLS, _RB * 4, 128), jnp.float32),
    mesh=_mesh,
    scratch_types=[
        pltpu.VMEM((2, _RB_W, 8, 128), jnp.int32),
        pltpu.VMEM((2, 8, _RB_W * 4, 128), jnp.float32),
        pltpu.VMEM((32,), jnp.float32),
        pltpu.SemaphoreType.DMA,
        pltpu.SemaphoreType.DMA,
        pltpu.SemaphoreType.DMA,
        pltpu.SemaphoreType.DMA,
    ],
    compiler_params=pltpu.CompilerParams(needs_layout_passes=False),
)
def _lookup(idx_hbm, tab_hbm, out_hbm, in_v, out_v, tab_v,
            isem0, isem1, osem0, osem1):
    wid = lax.axis_index("s") * _NC + lax.axis_index("c")
    rb0 = wid * _RB_W
    pltpu.sync_copy(tab_hbm, tab_v)
    isems = (isem0, isem1)
    osems = (osem0, osem1)

    def in_copy(cb, b):
        return pltpu.make_async_copy(
            idx_hbm.at[cb, pl.ds(rb0, _RB_W)], in_v.at[b], isems[b])

    def out_copy(cb, b):
        return pltpu.make_async_copy(
            out_v.at[b],
            out_hbm.at[pl.ds(cb * 8, 8), pl.ds(rb0 * 4, _RB_W * 4)],
            osems[b])

    def compute(b):
        @plsc.parallel_loop(0, 8 * _RB_W, unroll=4)
        def row_body(i):
            cs = i // _RB_W
            rbl = i - cs * _RB_W
            for g in range(8):
                iv = in_v[b, rbl, cs, pl.ds(g * 16, 16)]
                for j in range(4):
                    col = plsc.load_gather(tab_v, [iv + (6 * j)])
                    out_v[b, cs, rbl * 4 + j, pl.ds(g * 16, 16)] = col

    def unit(cb, b, skip_out_wait):
        in_copy(cb, b).wait()

        @pl.when(jnp.logical_not(skip_out_wait))
        def _():
            out_copy(cb, b).wait()

        compute(b)
        out_copy(cb, b).start()

    in_copy(0, 0).start()

    def pair_body(p, carry):
        cb0 = 2 * p
        in_copy(cb0 + 1, 1).start()
        unit(cb0, 0, p == 0)
        in_copy(cb0 + 2, 0).start()
        unit(cb0 + 1, 1, p == 0)
        return carry

    lax.fori_loop(0, (_CB - 1) // 2, pair_body, 0)
    unit(_CB - 1, 0, False)
    out_copy(_CB - 1, 0).wait()
    out_copy(_CB - 2, 1).wait()


def kernel(inputs, embed_weight):
    idx4 = (
        inputs.astype(jnp.int32)
        .reshape(_RB, 128, _CB, 8)
        .transpose(2, 0, 3, 1)
    )
    tab = jnp.zeros((32,), jnp.float32)
    tab = tab.at[:24].set(embed_weight.astype(jnp.float32).T.reshape(-1))
    out3 = _lookup(idx4, tab)
    return (
        out3.reshape(_COLS, _RB, 4, 128)
        .transpose(1, 3, 0, 2)
        .reshape(_ROWS, _COLS, 4)
    )
